# Initial kernel scaffold; baseline (speedup 1.0000x reference)
import jax
import jax.numpy as jnp
from jax.experimental import pallas as pl


def kernel(x, edge_index, edge_weight, W, b):
    raise NotImplementedError("write your pallas kernel here")


# speedup vs baseline: 3.9208x; 3.9208x over previous
import functools

import jax
import jax.numpy as jnp
from jax import lax
from jax.experimental import pallas as pl
from jax.experimental.pallas import tpu as pltpu
from jax.experimental.pallas import tpu_sc as plsc

N = 10000
D = 128
E = 320000
K = 2
NC, NS = 2, 16
NW = NC * NS
B = 128
NBLK = -(-E // (NW * B))
EPAD = NW * NBLK * B
NP = 10240
ROWS_PER_SUB = NP // NS


def _linear_relu(x, Wt, b2):
    def body(x_ref, w_ref, b_ref, o_ref):
        acc = jnp.dot(x_ref[...], w_ref[...], preferred_element_type=jnp.float32)
        o_ref[...] = jnp.maximum(acc + b_ref[...], 0.0)

    return pl.pallas_call(
        body,
        grid=(10,),
        in_specs=[
            pl.BlockSpec((N // 10, D), lambda i: (i, 0)),
            pl.BlockSpec((D, D), lambda i: (0, 0)),
            pl.BlockSpec((1, D), lambda i: (0, 0)),
        ],
        out_specs=pl.BlockSpec((N // 10, D), lambda i: (i, 0)),
        out_shape=jax.ShapeDtypeStruct((N, D), jnp.float32),
    )(x, Wt, b2)


def _add2(p):
    def body(p_ref, o_ref):
        o_ref[...] = p_ref[0] + p_ref[1]

    return pl.pallas_call(
        body,
        grid=(10,),
        in_specs=[pl.BlockSpec((2, N // 10, D), lambda i: (0, i, 0))],
        out_specs=pl.BlockSpec((N // 10, D), lambda i: (i, 0)),
        out_shape=jax.ShapeDtypeStruct((N, D), jnp.float32),
    )(p[:, :N])


def _propagate(h, src, dst, w, zeros_nd):
    mesh = plsc.VectorSubcoreMesh(core_axis_name="c", subcore_axis_name="s")

    @functools.partial(
        pl.kernel,
        out_type=jax.ShapeDtypeStruct((NC, NP, D), jnp.float32),
        mesh=mesh,
        scratch_types=[
            pltpu.VMEM((NBLK, B), jnp.int32),
            pltpu.VMEM((NBLK, B), jnp.int32),
            pltpu.VMEM((NBLK * B,), jnp.float32),
            pltpu.VMEM((B, D), jnp.float32),
            pltpu.VMEM_SHARED((NP, D), jnp.float32),
            pltpu.SemaphoreType.DMA,
        ],
    )
    def k(h_hbm, src_hbm, dst_hbm, w_hbm, z_hbm, out_hbm,
          src_v, dst_v, w_v, rows_v, acc_sh, sem):
        c = lax.axis_index("c")
        s = lax.axis_index("s")
        wid = c * NS + s

        pltpu.sync_copy(src_hbm.at[wid], src_v)
        pltpu.sync_copy(dst_hbm.at[wid], dst_v)
        pltpu.sync_copy(w_hbm.at[wid], w_v)

        row0 = s * ROWS_PER_SUB
        pltpu.sync_copy(z_hbm.at[pl.ds(row0, ROWS_PER_SUB)],
                        acc_sh.at[pl.ds(row0, ROWS_PER_SUB)])
        plsc.subcore_barrier()

        def blk(j, carry):
            pltpu.async_copy(h_hbm.at[src_v.at[j]], rows_v, sem).wait()

            def grp(eg, carry2):
                w16 = w_v[pl.ds(j * B + eg * 16, 16)]

                def edge(l, carry3):
                    e = eg * 16 + l
                    wv = lax.gather(
                        w16, jnp.full((16, 1), l, jnp.int32),
                        lax.GatherDimensionNumbers(
                            offset_dims=(), collapsed_slice_dims=(0,),
                            start_index_map=(0,)),
                        slice_sizes=(1,),
                        mode=lax.GatherScatterMode.PROMISE_IN_BOUNDS)
                    for g in range(D // 16):
                        rows_v[e, pl.ds(g * 16, 16)] = (
                            rows_v[e, pl.ds(g * 16, 16)] * wv)
                    return carry3

                return lax.fori_loop(0, 16, edge, carry2, unroll=False)

            lax.fori_loop(0, B // 16, grp, 0, unroll=False)
            pltpu.sync_copy(rows_v, acc_sh.at[dst_v.at[j]], add=True)
            return carry

        lax.fori_loop(0, NBLK, blk, 0, unroll=False)

        plsc.subcore_barrier()
        pltpu.sync_copy(acc_sh.at[pl.ds(row0, ROWS_PER_SUB)],
                        out_hbm.at[c, pl.ds(row0, ROWS_PER_SUB)])

    return k(h, src, dst, w, zeros_nd)


def kernel(x, edge_index, edge_weight, W, b):
    src = edge_index[0].astype(jnp.int32)
    dst = edge_index[1].astype(jnp.int32)
    w = edge_weight.astype(jnp.float32)

    pad = EPAD - E
    src = jnp.pad(src, (0, pad)).reshape(NW, NBLK, B)
    dst = jnp.pad(dst, (0, pad)).reshape(NW, NBLK, B)
    w = jnp.pad(w, (0, pad)).reshape(NW, NBLK * B)

    zeros_nd = jnp.zeros((NP, D), jnp.float32)

    h = _linear_relu(x, W.T, b.reshape(1, D))
    for _ in range(K):
        p = _propagate(h, src, dst, w, zeros_nd)
        h = _add2(p)
    return h
